# Initial kernel scaffold; baseline (speedup 1.0000x reference)
import jax
import jax.numpy as jnp
from jax.experimental import pallas as pl


def kernel(x, edge_index, batch, W1, a_src1, a_dst1, b1, W2, a_src2, a_dst2, b2, cls_W, cls_b):
    raise NotImplementedError("write your pallas kernel here")


# speedup vs baseline: 17.1344x; 17.1344x over previous
import functools

import jax
import jax.numpy as jnp
from jax import lax
from jax.experimental import pallas as pl
from jax.experimental.pallas import tpu as pltpu
from jax.experimental.pallas import tpu_sc as plsc

N = 10000
E = 320000
D = 128
H = 4
C = 64
HC = H * C
NCLS = 40
G = 64
NP = 10240
NSUB = 16
BLK = 64
EP_TILE = -(-(E + N) // (NSUB * BLK * 8)) * BLK * 8
NBLK = EP_TILE // BLK
CH = 8
EP = EP_TILE * NSUB
RB = 512
PW = 144
GW = 160
ADW = 32
EPS = 1e-16


def _bc16(col):
    return jnp.broadcast_to(col, (col.shape[0], 16))


def _emit_packed(h, al, i, packed_ref, ad_ref, stats_ref):
    nblk = pl.num_programs(0)
    for p in range(2):
        packed_ref[p, :, 0:128] = h[:, 128 * p:128 * p + 128]
        packed_ref[p, :, 128:144] = _bc16(al[:, 2 * p:2 * p + 1])
        packed_ref[p, :, 144:160] = _bc16(al[:, 2 * p + 1:2 * p + 2])
        ad_ref[p, :, 0:16] = _bc16(al[:, 16 + 2 * p:17 + 2 * p])
        ad_ref[p, :, 16:32] = _bc16(al[:, 17 + 2 * p:18 + 2 * p])
    cur = jnp.broadcast_to(jnp.max(al, axis=0, keepdims=True), (8, 128))

    @pl.when(i == 0)
    def _():
        stats_ref[...] = cur

    @pl.when(i > 0)
    def _():
        stats_ref[...] = jnp.maximum(stats_ref[...], cur)

    @pl.when(i == nblk - 1)
    def _():
        v = stats_ref[...]
        parts = []
        for hh in range(4):
            sh = v[0:1, hh:hh + 1] + v[0:1, 16 + hh:17 + hh]
            mh = jnp.maximum(sh, 0.2 * sh)
            parts.append(jnp.broadcast_to(mh, (1, 16)))
        parts.append(jnp.zeros((1, 64), jnp.float32))
        stats_ref[1:2, :] = jnp.concatenate(parts, axis=1)


def _prep1_body(x_ref, w_ref, a_ref, packed_ref, ad_ref, stats_ref):
    i = pl.program_id(0)
    h = jnp.dot(x_ref[...], w_ref[...], preferred_element_type=jnp.float32)
    al = jnp.dot(h, a_ref[...], preferred_element_type=jnp.float32)
    _emit_packed(h, al, i, packed_ref, ad_ref, stats_ref)


def _assemble_h(accA, accB, b):
    d00 = accA[:, 128:129] + EPS
    d01 = accA[:, 129:130] + EPS
    d10 = accB[:, 128:129] + EPS
    d11 = accB[:, 129:130] + EPS
    hin = jnp.concatenate([
        accA[:, 0:64] / d00, accA[:, 64:128] / d01,
        accB[:, 0:64] / d10, accB[:, 64:128] / d11,
    ], axis=1)
    return jnp.maximum(hin + b, 0.0)


def _mid_body(accA_ref, accB_ref, b_ref, w_ref, a_ref, packed_ref, ad_ref, stats_ref):
    i = pl.program_id(0)
    hin = _assemble_h(accA_ref[...], accB_ref[...], b_ref[...])
    h = jnp.dot(hin, w_ref[...], preferred_element_type=jnp.float32)
    al = jnp.dot(h, a_ref[...], preferred_element_type=jnp.float32)
    _emit_packed(h, al, i, packed_ref, ad_ref, stats_ref)


def _final_body(accA_ref, accB_ref, b_ref, batch_ref, clsw_ref, clsb_ref,
                out_ref, pooled_scr, cnt_scr):
    i = pl.program_id(0)
    nblk = pl.num_programs(0)
    h2 = _assemble_h(accA_ref[...], accB_ref[...], b_ref[...])
    g_iota = lax.broadcasted_iota(jnp.int32, (h2.shape[0], G), 1)
    oh = (batch_ref[...] == g_iota).astype(jnp.float32)
    psum = lax.dot_general(oh, h2, (((0,), (0,)), ((), ())),
                           preferred_element_type=jnp.float32)
    csum = lax.dot_general(oh, jnp.ones_like(h2[:, 0:128]),
                           (((0,), (0,)), ((), ())),
                           preferred_element_type=jnp.float32)

    @pl.when(i == 0)
    def _():
        pooled_scr[...] = psum
        cnt_scr[...] = csum

    @pl.when(i > 0)
    def _():
        pooled_scr[...] = pooled_scr[...] + psum
        cnt_scr[...] = cnt_scr[...] + csum

    @pl.when(i == nblk - 1)
    def _():
        pooled = pooled_scr[...] / jnp.maximum(cnt_scr[:, 0:1], 1.0)
        out_ref[...] = jnp.dot(pooled, clsw_ref[...],
                               preferred_element_type=jnp.float32) + clsb_ref[...]


_SDS = jax.ShapeDtypeStruct
_GRID = NP // RB


def _tc_prep1(x_pad, W1, A1):
    return pl.pallas_call(
        _prep1_body,
        grid=(_GRID,),
        in_specs=[
            pl.BlockSpec((RB, D), lambda i: (i, 0)),
            pl.BlockSpec((D, HC), lambda i: (0, 0)),
            pl.BlockSpec((HC, 128), lambda i: (0, 0)),
        ],
        out_specs=[
            pl.BlockSpec((2, RB, GW), lambda i: (0, i, 0)),
            pl.BlockSpec((2, RB, ADW), lambda i: (0, i, 0)),
            pl.BlockSpec((8, 128), lambda i: (0, 0)),
        ],
        out_shape=[
            _SDS((2, NP, GW), jnp.float32),
            _SDS((2, NP, ADW), jnp.float32),
            _SDS((8, 128), jnp.float32),
        ],
    )(x_pad, W1, A1)


def _tc_mid(accA, accB, b1, W2, A2):
    return pl.pallas_call(
        _mid_body,
        grid=(_GRID,),
        in_specs=[
            pl.BlockSpec((RB, PW), lambda i: (i, 0)),
            pl.BlockSpec((RB, PW), lambda i: (i, 0)),
            pl.BlockSpec((1, HC), lambda i: (0, 0)),
            pl.BlockSpec((HC, HC), lambda i: (0, 0)),
            pl.BlockSpec((HC, 128), lambda i: (0, 0)),
        ],
        out_specs=[
            pl.BlockSpec((2, RB, GW), lambda i: (0, i, 0)),
            pl.BlockSpec((2, RB, ADW), lambda i: (0, i, 0)),
            pl.BlockSpec((8, 128), lambda i: (0, 0)),
        ],
        out_shape=[
            _SDS((2, NP, GW), jnp.float32),
            _SDS((2, NP, ADW), jnp.float32),
            _SDS((8, 128), jnp.float32),
        ],
    )(accA, accB, b1, W2, A2)


def _tc_final(accA, accB, b2, batch2d, clsWp, clsbp):
    return pl.pallas_call(
        _final_body,
        grid=(_GRID,),
        in_specs=[
            pl.BlockSpec((RB, PW), lambda i: (i, 0)),
            pl.BlockSpec((RB, PW), lambda i: (i, 0)),
            pl.BlockSpec((1, HC), lambda i: (0, 0)),
            pl.BlockSpec((RB, 1), lambda i: (i, 0)),
            pl.BlockSpec((HC, 128), lambda i: (0, 0)),
            pl.BlockSpec((1, 128), lambda i: (0, 0)),
        ],
        out_specs=pl.BlockSpec((G, 128), lambda i: (0, 0)),
        out_shape=_SDS((G, 128), jnp.float32),
        scratch_shapes=[
            pltpu.VMEM((G, HC), jnp.float32),
            pltpu.VMEM((G, 128), jnp.float32),
        ],
    )(accA, accB, b2, batch2d, clsWp, clsbp)


def _full16(v):
    return jnp.broadcast_to(jnp.asarray(v, jnp.int32), (16,))


def _sc_body(src_hbm, dst_hbm, pA, pB, adA, adB, stats_hbm, out_hbm,
             acc, srcb, dstb, rows, msg, adv, statv,
             sem1, sem2):
    c = lax.axis_index("c")
    s = lax.axis_index("s")
    iota16 = lax.broadcasted_iota(jnp.int32, (16,), 0)
    zeros16 = jnp.zeros((16,), jnp.float32)

    def _z(k, _):
        r = k // 9
        j = k - r * 9
        msg[r, pl.ds(j * 16, 16)] = zeros16
        return 0
    lax.fori_loop(0, BLK * 9, _z, 0)
    for k in range(NP // NSUB // BLK):
        pltpu.sync_copy(msg, acc.at[pl.ds(s * (NP // NSUB) + k * BLK, BLK)])

    pltpu.sync_copy(stats_hbm, statv)
    m0 = statv[pl.ds(c * 32, 16)]
    m1 = statv[pl.ds(c * 32 + 16, 16)]

    plsc.subcore_barrier()

    oh0 = jnp.where(iota16 == 0, 1.0, 0.0)
    oh1 = jnp.where(iota16 == 1, 1.0, 0.0)

    def _edge(e, _):
        as0 = rows[e, pl.ds(128, 16)]
        as1 = rows[e, pl.ds(144, 16)]
        ad0 = adv[e, pl.ds(0, 16)]
        ad1 = adv[e, pl.ds(16, 16)]
        t0 = as0 + ad0
        t0 = jnp.maximum(t0, 0.2 * t0)
        w0 = jnp.exp(t0 - m0)
        t1 = as1 + ad1
        t1 = jnp.maximum(t1, 0.2 * t1)
        w1 = jnp.exp(t1 - m1)
        for j in range(4):
            msg[e, pl.ds(j * 16, 16)] = rows[e, pl.ds(j * 16, 16)] * w0
        for j in range(4, 8):
            msg[e, pl.ds(j * 16, 16)] = rows[e, pl.ds(j * 16, 16)] * w1
        msg[e, pl.ds(128, 16)] = w0 * oh0 + w1 * oh1
        return 0

    def _chunk(ch, _):
        pltpu.sync_copy(src_hbm.at[pl.ds(s * NBLK + ch * CH, CH)], srcb)
        pltpu.sync_copy(dst_hbm.at[pl.ds(s * NBLK + ch * CH, CH)], dstb)

        def _blk(gg, _):
            @pl.when(c == 0)
            def _():
                d1 = pltpu.async_copy(pA.at[srcb.at[gg]], rows, sem1)
                d2 = pltpu.async_copy(adA.at[dstb.at[gg]], adv, sem2)
                d1.wait()
                d2.wait()

            @pl.when(c != 0)
            def _():
                d1 = pltpu.async_copy(pB.at[srcb.at[gg]], rows, sem1)
                d2 = pltpu.async_copy(adB.at[dstb.at[gg]], adv, sem2)
                d1.wait()
                d2.wait()

            lax.fori_loop(0, BLK, _edge, 0)
            pltpu.sync_copy(msg, acc.at[dstb.at[gg]], add=True)
            return 0

        lax.fori_loop(0, CH, _blk, 0)
        return 0

    lax.fori_loop(0, NBLK // CH, _chunk, 0)
    plsc.subcore_barrier()

    for k in range(NP // NSUB // BLK):
        base = s * (NP // NSUB) + k * BLK
        pltpu.sync_copy(acc.at[pl.ds(base, BLK)], out_hbm.at[c, pl.ds(base, BLK)])


@functools.cache
def _sc_edge():
    return pl.kernel(
        _sc_body,
        out_type=_SDS((2, NP, PW), jnp.float32),
        mesh=plsc.VectorSubcoreMesh(core_axis_name="c", subcore_axis_name="s",
                                    num_cores=2, num_subcores=NSUB),
        compiler_params=pltpu.CompilerParams(use_tc_tiling_on_sc=False),
        scratch_types=[
            pltpu.VMEM_SHARED((NP, PW), jnp.float32),
            pltpu.VMEM((CH, BLK), jnp.int32),
            pltpu.VMEM((CH, BLK), jnp.int32),
            pltpu.VMEM((BLK, GW), jnp.float32),
            pltpu.VMEM((BLK, PW), jnp.float32),
            pltpu.VMEM((BLK, ADW), jnp.float32),
            pltpu.VMEM((128,), jnp.float32),
            pltpu.SemaphoreType.DMA,
            pltpu.SemaphoreType.DMA,
        ],
    )


def _proj_mat(a_src, a_dst):
    eye = jnp.eye(H, dtype=jnp.float32)
    xs = (a_src[:, :, None] * eye[:, None, :]).reshape(HC, H)
    xd = (a_dst[:, :, None] * eye[:, None, :]).reshape(HC, H)
    z12 = jnp.zeros((HC, 16 - H), jnp.float32)
    return jnp.concatenate([xs, z12, xd, jnp.zeros((HC, 128 - 16 - H), jnp.float32)],
                           axis=1)


def kernel(x, edge_index, batch, W1, a_src1, a_dst1, b1, W2, a_src2, a_dst2, b2,
           cls_W, cls_b):
    npad = EP - (E + N)
    loop = jnp.arange(N, dtype=jnp.int32)
    pad_src = (jnp.arange(npad, dtype=jnp.int32) * 131) % N
    pad_dst = N + (jnp.arange(npad, dtype=jnp.int32) % 16)
    src = jnp.concatenate([edge_index[0], loop, pad_src]).reshape(EP // BLK, BLK)
    dst = jnp.concatenate([edge_index[1], loop, pad_dst]).reshape(EP // BLK, BLK)

    x_pad = jnp.pad(x, ((0, NP - N), (0, 0)))
    A1 = _proj_mat(a_src1, a_dst1)
    A2 = _proj_mat(a_src2, a_dst2)
    batch2d = jnp.pad(batch, (0, NP - N), constant_values=G).reshape(NP, 1)
    clsWp = jnp.pad(cls_W, ((0, 0), (0, 128 - NCLS)))
    clsbp = jnp.pad(cls_b, (0, 128 - NCLS)).reshape(1, 128)

    packed, ad, stats = _tc_prep1(x_pad, W1, A1)
    acc = _sc_edge()(src, dst, packed[0], packed[1], ad[0], ad[1], stats[1])

    packed, ad, stats = _tc_mid(acc[0], acc[1], b1.reshape(1, HC), W2, A2)
    acc = _sc_edge()(src, dst, packed[0], packed[1], ad[0], ad[1], stats[1])

    out = _tc_final(acc[0], acc[1], b2.reshape(1, HC), batch2d, clsWp, clsbp)
    return out[:, :NCLS]
